# Initial kernel scaffold; baseline (speedup 1.0000x reference)
import functools

import jax
import jax.numpy as jnp
from jax import lax
from jax.experimental import pallas as pl
from jax.experimental.pallas import tpu as pltpu
from jax.experimental.pallas import tpu_sc as plsc

NC = 2
NS = 16
L = 16
NW = NC * NS
CHUNK = 128


def _pre_body(logits_ref, nf_ref, wp_ref, bp_ref, ex_ref, hv_ref):
    l = logits_ref[...]
    m = jnp.max(l)
    ex_ref[...] = jnp.exp(l - m)
    hv_ref[...] = (
        jnp.dot(nf_ref[...], wp_ref[...], preferred_element_type=jnp.float32)
        + bp_ref[...]
    )


def _make_agg(v_rows, v_pad, kch):
    rps = v_pad // NS

    mesh = plsc.VectorSubcoreMesh(core_axis_name="c", subcore_axis_name="s")

    @functools.partial(
        pl.kernel,
        mesh=mesh,
        out_type=[
            jax.ShapeDtypeStruct((NC, v_pad, 128), jnp.float32),
            jax.ShapeDtypeStruct((NC, v_pad), jnp.float32),
        ],
        scratch_types=[
            pltpu.VMEM((kch, CHUNK), jnp.int32),
            pltpu.VMEM((kch, CHUNK), jnp.int32),
            pltpu.VMEM((kch, CHUNK), jnp.float32),
            pltpu.VMEM((CHUNK, 128), jnp.float32),
            pltpu.VMEM((CHUNK, 128), jnp.float32),
            pltpu.VMEM((v_pad // NS,), jnp.float32),
            pltpu.VMEM_SHARED((v_pad, 128), jnp.float32),
            pltpu.VMEM_SHARED((v_pad,), jnp.float32),
            pltpu.SemaphoreType.DMA,
        ],
    )
    def agg(hv_hbm, src_hbm, dst_hbm, ex_hbm, cm_out, s_out,
            src_v, dst_v, ex_v, rows_v, zero_v, zs_v, cm_sh, s_sh, sem):
        c = lax.axis_index("c")
        s = lax.axis_index("s")
        wid = s * NC + c

        zv = jnp.zeros((L,), jnp.float32)

        def zrow(i, carry):
            for j in range(128 // L):
                zero_v[i, pl.ds(j * L, L)] = zv
            return carry

        lax.fori_loop(0, CHUNK, zrow, 0)

        def zsrow(i, carry):
            zs_v[pl.ds(i * L, L)] = zv
            return carry

        lax.fori_loop(0, rps // L, zsrow, 0)

        for i in range(rps // CHUNK):
            pltpu.sync_copy(zero_v, cm_sh.at[pl.ds(s * rps + i * CHUNK, CHUNK)])
        pltpu.sync_copy(zs_v, s_sh.at[pl.ds(s * rps, rps)])

        pltpu.sync_copy(src_hbm.at[wid], src_v)
        pltpu.sync_copy(dst_hbm.at[wid], dst_v)
        pltpu.sync_copy(ex_hbm.at[wid], ex_v)

        plsc.subcore_barrier()

        def chunk_body(k, carry):
            pltpu.async_copy(hv_hbm.at[src_v.at[k]], rows_v, sem).wait()

            def scale_g(g, carry2):
                evec = ex_v[k, pl.ds(g * L, L)]
                rid = g * L + lax.iota(jnp.int32, L)
                for j in range(128):
                    cid = jnp.full((L,), j, jnp.int32)
                    col = plsc.load_gather(rows_v, [rid, cid])
                    plsc.store_scatter(rows_v, [rid, cid], col * evec)
                return carry2

            lax.fori_loop(0, CHUNK // L, scale_g, 0)

            pltpu.sync_copy(rows_v, cm_sh.at[dst_v.at[k]], add=True)
            pltpu.sync_copy(ex_v.at[k], s_sh.at[dst_v.at[k]], add=True)
            return carry

        lax.fori_loop(0, kch, chunk_body, 0)

        plsc.subcore_barrier()

        pltpu.sync_copy(cm_sh.at[pl.ds(s * rps, rps)],
                        cm_out.at[c].at[pl.ds(s * rps, rps)])
        pltpu.sync_copy(s_sh.at[pl.ds(s * rps, rps)],
                        s_out.at[c].at[pl.ds(s * rps, rps)])

    return agg


def _gru_body(cm_ref, s_ref, nf_ref, wih_ref, bih_ref, whh_ref, bhh_ref, o_ref):
    cm = cm_ref[0] + cm_ref[1]
    sv = s_ref[0] + s_ref[1]
    pos = sv > 0.0
    ctx = jnp.where(pos, cm / jnp.where(pos, sv, 1.0), 0.0)
    ctx = jnp.where(ctx > 0.0, ctx, jnp.exp(ctx) - 1.0)
    nf = nf_ref[...]
    gi = jnp.dot(ctx, wih_ref[...], preferred_element_type=jnp.float32) + bih_ref[...]
    gh = jnp.dot(nf, whh_ref[...], preferred_element_type=jnp.float32) + bhh_ref[...]
    d = o_ref.shape[1]
    r = jax.nn.sigmoid(gi[:, :d] + gh[:, :d])
    z = jax.nn.sigmoid(gi[:, d:2 * d] + gh[:, d:2 * d])
    n = jnp.tanh(gi[:, 2 * d:] + r * gh[:, 2 * d:])
    o_ref[...] = jnp.maximum((1.0 - z) * n + z * nf, 0.0)


def kernel(edge_index, edge_logits, node_feats, W_proj, b_proj, W_ih, b_ih,
           W_hh, b_hh):
    V, D = node_feats.shape
    H = W_proj.shape[0]
    E = edge_index.shape[1]

    ew = NW * CHUNK
    e_pad = -(-E // ew) * ew
    kch = e_pad // ew
    rps = -(-V // (NS * CHUNK)) * CHUNK
    v_pad = NS * rps

    src = jnp.concatenate(
        [edge_index[0].astype(jnp.int32), jnp.zeros((e_pad - E,), jnp.int32)]
    ).reshape(NW, kch, CHUNK)
    dst = jnp.concatenate(
        [edge_index[1].astype(jnp.int32), jnp.zeros((e_pad - E,), jnp.int32)]
    ).reshape(NW, kch, CHUNK)
    logits = jnp.concatenate(
        [edge_logits[:, 0], jnp.full((e_pad - E,), -jnp.inf, jnp.float32)]
    ).reshape(e_pad // 128, 128)

    ex, hv = pl.pallas_call(
        _pre_body,
        out_shape=[
            jax.ShapeDtypeStruct((e_pad // 128, 128), jnp.float32),
            jax.ShapeDtypeStruct((V, H), jnp.float32),
        ],
    )(logits, node_feats, W_proj.T, b_proj.reshape(1, H))

    agg = _make_agg(V, v_pad, kch)
    cm, sacc = agg(hv, src, dst, ex.reshape(NW, kch, CHUNK))

    nf_pad = jnp.concatenate(
        [node_feats, jnp.zeros((v_pad - V, D), jnp.float32)]
    )
    bv = rps
    grid = v_pad // bv
    out = pl.pallas_call(
        _gru_body,
        grid=(grid,),
        in_specs=[
            pl.BlockSpec((NC, bv, H), lambda i: (0, i, 0)),
            pl.BlockSpec((NC, bv, 1), lambda i: (0, i, 0)),
            pl.BlockSpec((bv, D), lambda i: (i, 0)),
            pl.BlockSpec((H, 3 * D), lambda i: (0, 0)),
            pl.BlockSpec((1, 3 * D), lambda i: (0, 0)),
            pl.BlockSpec((D, 3 * D), lambda i: (0, 0)),
            pl.BlockSpec((1, 3 * D), lambda i: (0, 0)),
        ],
        out_specs=pl.BlockSpec((bv, D), lambda i: (i, 0)),
        out_shape=jax.ShapeDtypeStruct((v_pad, D), jnp.float32),
    )(cm, sacc.reshape(NC, v_pad, 1), nf_pad, W_ih.T, b_ih.reshape(1, 3 * D),
      W_hh.T, b_hh.reshape(1, 3 * D))

    return out[:V]


# speedup vs baseline: 10.7865x; 10.7865x over previous
import functools

import jax
import jax.numpy as jnp
from jax import lax
from jax.experimental import pallas as pl
from jax.experimental.pallas import tpu as pltpu
from jax.experimental.pallas import tpu_sc as plsc

NC = 2
NS = 16
L = 16
NW = NC * NS
CHUNK = 128


def _pre_body(logits_ref, nf_ref, wp_ref, bp_ref, ex_ref, hv_ref):
    l = logits_ref[...]
    m = jnp.max(l)
    ex_ref[...] = jnp.exp(l - m)
    hv_ref[...] = (
        jnp.dot(nf_ref[...], wp_ref[...], preferred_element_type=jnp.float32)
        + bp_ref[...]
    )


def _make_agg(v_rows, v_pad, kch):
    rps = v_pad // NS

    mesh = plsc.VectorSubcoreMesh(core_axis_name="c", subcore_axis_name="s")

    @functools.partial(
        pl.kernel,
        mesh=mesh,
        out_type=[
            jax.ShapeDtypeStruct((NC, v_pad, 128), jnp.float32),
            jax.ShapeDtypeStruct((NC, v_pad), jnp.float32),
        ],
        scratch_types=[
            pltpu.VMEM((kch, CHUNK), jnp.int32),
            pltpu.VMEM((kch, CHUNK), jnp.int32),
            pltpu.VMEM((kch, CHUNK), jnp.float32),
            pltpu.VMEM((CHUNK, 128), jnp.float32),
            pltpu.VMEM((v_pad // NS,), jnp.float32),
            pltpu.VMEM_SHARED((v_pad, 128), jnp.float32),
            pltpu.VMEM_SHARED((v_pad,), jnp.float32),
            pltpu.SemaphoreType.DMA,
        ],
    )
    def agg(hv_hbm, src_hbm, dst_hbm, ex_hbm, cm_out, s_out,
            src_v, dst_v, ex_v, rows_v, zs_v, cm_sh, s_sh, sem):
        c = lax.axis_index("c")
        s = lax.axis_index("s")
        wid = s * NC + c

        zv = jnp.zeros((L,), jnp.float32)

        def zrow(i, carry):
            for j in range(128 // L):
                rows_v[i, pl.ds(j * L, L)] = zv
            return carry

        lax.fori_loop(0, CHUNK, zrow, 0)

        def zsrow(i, carry):
            zs_v[pl.ds(i * L, L)] = zv
            return carry

        lax.fori_loop(0, rps // L, zsrow, 0)

        for i in range(rps // CHUNK):
            pltpu.sync_copy(rows_v, cm_sh.at[pl.ds(s * rps + i * CHUNK, CHUNK)])
        pltpu.sync_copy(zs_v, s_sh.at[pl.ds(s * rps, rps)])

        pltpu.sync_copy(src_hbm.at[wid], src_v)
        pltpu.sync_copy(dst_hbm.at[wid], dst_v)
        pltpu.sync_copy(ex_hbm.at[wid], ex_v)

        plsc.subcore_barrier()

        def chunk_body(k, carry):
            pltpu.async_copy(hv_hbm.at[src_v.at[k]], rows_v, sem).wait()

            def scale_g(g, carry2):
                base = g * L
                evec = ex_v[k, pl.ds(base, L)]
                for i in range(L):
                    e = evec[i]
                    for j in range(128 // L):
                        sl = pl.ds(j * L, L)
                        rows_v[base + i, sl] = rows_v[base + i, sl] * e
                return carry2

            lax.fori_loop(0, CHUNK // L, scale_g, 0)

            pltpu.sync_copy(rows_v, cm_sh.at[dst_v.at[k]], add=True)
            pltpu.sync_copy(ex_v.at[k], s_sh.at[dst_v.at[k]], add=True)
            return carry

        lax.fori_loop(0, kch, chunk_body, 0)

        plsc.subcore_barrier()

        pltpu.sync_copy(cm_sh.at[pl.ds(s * rps, rps)],
                        cm_out.at[c].at[pl.ds(s * rps, rps)])
        pltpu.sync_copy(s_sh.at[pl.ds(s * rps, rps)],
                        s_out.at[c].at[pl.ds(s * rps, rps)])

    return agg


def _gru_body(cm_ref, s_ref, nf_ref, wih_ref, bih_ref, whh_ref, bhh_ref, o_ref):
    cm = cm_ref[0] + cm_ref[1]
    sv = s_ref[0] + s_ref[1]
    pos = sv > 0.0
    ctx = jnp.where(pos, cm / jnp.where(pos, sv, 1.0), 0.0)
    ctx = jnp.where(ctx > 0.0, ctx, jnp.exp(ctx) - 1.0)
    nf = nf_ref[...]
    gi = jnp.dot(ctx, wih_ref[...], preferred_element_type=jnp.float32) + bih_ref[...]
    gh = jnp.dot(nf, whh_ref[...], preferred_element_type=jnp.float32) + bhh_ref[...]
    d = o_ref.shape[1]
    r = jax.nn.sigmoid(gi[:, :d] + gh[:, :d])
    z = jax.nn.sigmoid(gi[:, d:2 * d] + gh[:, d:2 * d])
    n = jnp.tanh(gi[:, 2 * d:] + r * gh[:, 2 * d:])
    o_ref[...] = jnp.maximum((1.0 - z) * n + z * nf, 0.0)


def kernel(edge_index, edge_logits, node_feats, W_proj, b_proj, W_ih, b_ih,
           W_hh, b_hh):
    V, D = node_feats.shape
    H = W_proj.shape[0]
    E = edge_index.shape[1]

    ew = NW * CHUNK
    e_pad = -(-E // ew) * ew
    kch = e_pad // ew
    rps = -(-V // (NS * CHUNK)) * CHUNK
    v_pad = NS * rps

    src = jnp.concatenate(
        [edge_index[0].astype(jnp.int32), jnp.zeros((e_pad - E,), jnp.int32)]
    ).reshape(NW, kch, CHUNK)
    dst = jnp.concatenate(
        [edge_index[1].astype(jnp.int32), jnp.zeros((e_pad - E,), jnp.int32)]
    ).reshape(NW, kch, CHUNK)
    logits = jnp.concatenate(
        [edge_logits[:, 0], jnp.full((e_pad - E,), -jnp.inf, jnp.float32)]
    ).reshape(e_pad // 128, 128)

    ex, hv = pl.pallas_call(
        _pre_body,
        out_shape=[
            jax.ShapeDtypeStruct((e_pad // 128, 128), jnp.float32),
            jax.ShapeDtypeStruct((V, H), jnp.float32),
        ],
    )(logits, node_feats, W_proj.T, b_proj.reshape(1, H))

    agg = _make_agg(V, v_pad, kch)
    cm, sacc = agg(hv, src, dst, ex.reshape(NW, kch, CHUNK))

    nf_pad = jnp.concatenate(
        [node_feats, jnp.zeros((v_pad - V, D), jnp.float32)]
    )
    bv = rps
    grid = v_pad // bv
    out = pl.pallas_call(
        _gru_body,
        grid=(grid,),
        in_specs=[
            pl.BlockSpec((NC, bv, H), lambda i: (0, i, 0)),
            pl.BlockSpec((NC, bv, 1), lambda i: (0, i, 0)),
            pl.BlockSpec((bv, D), lambda i: (i, 0)),
            pl.BlockSpec((H, 3 * D), lambda i: (0, 0)),
            pl.BlockSpec((1, 3 * D), lambda i: (0, 0)),
            pl.BlockSpec((D, 3 * D), lambda i: (0, 0)),
            pl.BlockSpec((1, 3 * D), lambda i: (0, 0)),
        ],
        out_specs=pl.BlockSpec((bv, D), lambda i: (i, 0)),
        out_shape=jax.ShapeDtypeStruct((v_pad, D), jnp.float32),
    )(cm, sacc.reshape(NC, v_pad, 1), nf_pad, W_ih.T, b_ih.reshape(1, 3 * D),
      W_hh.T, b_hh.reshape(1, 3 * D))

    return out[:V]
